# Initial kernel scaffold; baseline (speedup 1.0000x reference)
import jax
import jax.numpy as jnp
from jax.experimental import pallas as pl


def kernel(input_, neighbors, max_iter):
    raise NotImplementedError("write your pallas kernel here")


# speedup vs baseline: 7.6951x; 7.6951x over previous
import functools

import jax
import jax.numpy as jnp
from jax import lax
from jax.experimental import pallas as pl
from jax.experimental.pallas import tpu as pltpu
from jax.experimental.pallas import tpu_sc as plsc

DIM = 256
K = 8
DD = DIM // K

NC = 2
NS = 16
NW = NC * NS
CH = 80


def _seg_matrix():
    r = lax.broadcasted_iota(jnp.int32, (DIM, K), 0) // DD
    c = lax.broadcasted_iota(jnp.int32, (DIM, K), 1)
    return (r == c).astype(jnp.float32)


def _seg_matrix_t():
    r = lax.broadcasted_iota(jnp.int32, (K, DIM), 0)
    c = lax.broadcasted_iota(jnp.int32, (K, DIM), 1) // DD
    return (r == c).astype(jnp.float32)


def _dot8(t, w):
    return lax.dot_general(t, w, (((1,), (0,)), ((), ())),
                           preferred_element_type=jnp.float32)


def _norm_body(x_ref, o_ref):
    x = x_ref[...]
    w = _seg_matrix()
    wt = _seg_matrix_t()
    n2 = _dot8(x * x, w)
    inv = 1.0 / jnp.maximum(jnp.sqrt(n2), 1e-12)
    o_ref[...] = x * _dot8(inv, wt)


def _normalize(x):
    n = x.shape[0]
    bn = 2000
    return pl.pallas_call(
        _norm_body,
        grid=(n // bn,),
        in_specs=[pl.BlockSpec((bn, DIM), lambda i: (i, 0))],
        out_specs=pl.BlockSpec((bn, DIM), lambda i: (i, 0)),
        out_shape=jax.ShapeDtypeStruct((n, DIM), jnp.float32),
    )(x)


def _gather(z, idx):
    e_total = idx.shape[0]
    n_chunks = e_total // CH
    t_max = (n_chunks + NW - 1) // NW
    mesh = plsc.VectorSubcoreMesh(
        core_axis_name="c", subcore_axis_name="s",
        num_cores=NC, num_subcores=NS)

    @functools.partial(
        pl.kernel,
        mesh=mesh,
        out_type=jax.ShapeDtypeStruct((e_total, DIM), jnp.float32),
        scratch_types=[
            pltpu.VMEM((CH,), jnp.int32),
            pltpu.VMEM((CH, DIM), jnp.float32),
            pltpu.SemaphoreType.DMA,
        ],
    )
    def k(z_hbm, idx_hbm, out_hbm, idx_v, rows_v, sem):
        wid = lax.axis_index("s") * NC + lax.axis_index("c")

        def body(t, carry):
            cid = t * NW + wid

            @pl.when(cid < n_chunks)
            def _():
                base = cid * CH
                pltpu.sync_copy(idx_hbm.at[pl.ds(base, CH)], idx_v)
                pltpu.async_copy(z_hbm.at[idx_v], rows_v, sem).wait()
                pltpu.sync_copy(rows_v, out_hbm.at[pl.ds(base, CH)])

            return carry

        lax.fori_loop(0, t_max, body, 0)

    return k(z, idx)


def _route_body(mi_ref, x_ref, zg_ref, o_ref, *, bn):
    w = _seg_matrix()
    wt = _seg_matrix_t()
    x = x_ref[...]
    zg0 = zg_ref[:, :DIM]
    zg1 = zg_ref[:, DIM:]
    mi = mi_ref[0]

    def body(it, u):
        s0 = _dot8(zg0 * u, w)
        s1 = _dot8(zg1 * u, w)
        e0 = jnp.exp(s0)
        e1 = jnp.exp(s1)
        p0 = e0 / jnp.sum(e0, axis=1, keepdims=True)
        p1 = e1 / jnp.sum(e1, axis=1, keepdims=True)
        u2 = zg0 * _dot8(p0, wt) + zg1 * _dot8(p1, wt) + x
        n2 = _dot8(u2 * u2, w)
        inv = 1.0 / jnp.maximum(jnp.sqrt(n2), 1e-12)
        un = u2 * _dot8(inv, wt)
        return jnp.where(it < mi - 1, un, u2)

    u0 = jnp.zeros((bn, DIM), jnp.float32)
    o_ref[...] = lax.fori_loop(0, mi, body, u0)


def _routing(z, zg2, mi_arr):
    n = z.shape[0]
    bn = 1000
    return pl.pallas_call(
        functools.partial(_route_body, bn=bn),
        grid=(n // bn,),
        in_specs=[
            pl.BlockSpec(memory_space=pltpu.SMEM),
            pl.BlockSpec((bn, DIM), lambda i: (i, 0)),
            pl.BlockSpec((bn, 2 * DIM), lambda i: (i, 0)),
        ],
        out_specs=pl.BlockSpec((bn, DIM), lambda i: (i, 0)),
        out_shape=jax.ShapeDtypeStruct((n, DIM), jnp.float32),
    )(mi_arr, z, zg2)


def kernel(input_, neighbors, max_iter):
    a, b, d = input_.shape
    n = a * b
    x = input_.reshape(n, d)
    z = _normalize(x)
    e_total = neighbors.shape[0]
    idx = neighbors.astype(jnp.int32) + (
        jnp.arange(e_total, dtype=jnp.int32) // 20) * 5
    zg = _gather(z, idx)
    zg2 = zg.reshape(n, 2 * d)
    mi_arr = jnp.asarray(max_iter, dtype=jnp.int32).reshape(1)
    u = _routing(z, zg2, mi_arr)
    return u.reshape(a, b, d)


# speedup vs baseline: 9.0001x; 1.1696x over previous
import functools

import jax
import jax.numpy as jnp
from jax import lax
from jax.experimental import pallas as pl
from jax.experimental.pallas import tpu as pltpu
from jax.experimental.pallas import tpu_sc as plsc

DIM = 256
K = 8
DD = DIM // K

NC = 2
NS = 16
NW = NC * NS
CH = 80


def _seg_matrix():
    r = lax.broadcasted_iota(jnp.int32, (DIM, K), 0) // DD
    c = lax.broadcasted_iota(jnp.int32, (DIM, K), 1)
    return (r == c).astype(jnp.float32)


def _seg_matrix_t():
    r = lax.broadcasted_iota(jnp.int32, (K, DIM), 0)
    c = lax.broadcasted_iota(jnp.int32, (K, DIM), 1) // DD
    return (r == c).astype(jnp.float32)


def _dot8(t, w):
    return lax.dot_general(t, w, (((1,), (0,)), ((), ())),
                           preferred_element_type=jnp.float32)


def _normseg(t, w, wt):
    n2 = _dot8(t * t, w)
    inv = 1.0 / jnp.maximum(jnp.sqrt(n2), 1e-12)
    return t * _dot8(inv, wt)


def _norm_body(x_ref, o_ref):
    o_ref[...] = _normseg(x_ref[...], _seg_matrix(), _seg_matrix_t())


def _normalize(x):
    n = x.shape[0]
    bn = 4000
    return pl.pallas_call(
        _norm_body,
        grid=(n // bn,),
        in_specs=[pl.BlockSpec((bn, DIM), lambda i: (i, 0))],
        out_specs=pl.BlockSpec((bn, DIM), lambda i: (i, 0)),
        out_shape=jax.ShapeDtypeStruct((n, DIM), jnp.float32),
    )(x)


def _gather(z, idx):
    e_total = idx.shape[0]
    n_chunks = e_total // CH
    t_max = (n_chunks + NW - 1) // NW
    mesh = plsc.VectorSubcoreMesh(
        core_axis_name="c", subcore_axis_name="s",
        num_cores=NC, num_subcores=NS)

    @functools.partial(
        pl.kernel,
        mesh=mesh,
        out_type=jax.ShapeDtypeStruct((e_total, DIM), jnp.float32),
        scratch_types=[
            pltpu.VMEM((CH,), jnp.int32),
            pltpu.VMEM((CH, DIM), jnp.float32),
            pltpu.SemaphoreType.DMA,
        ],
    )
    def k(z_hbm, idx_hbm, out_hbm, idx_v, rows_v, sem):
        wid = lax.axis_index("s") * NC + lax.axis_index("c")

        def body(t, carry):
            cid = t * NW + wid

            @pl.when(cid < n_chunks)
            def _():
                base = cid * CH
                pltpu.sync_copy(idx_hbm.at[pl.ds(base, CH)], idx_v)
                pltpu.async_copy(z_hbm.at[idx_v], rows_v, sem).wait()
                pltpu.sync_copy(rows_v, out_hbm.at[pl.ds(base, CH)])

            return carry

        lax.fori_loop(0, t_max, body, 0)

    return k(z, idx)


def _route_body(mi_ref, x_ref, zg_ref, o_ref, *, bn):
    w = _seg_matrix()
    wt = _seg_matrix_t()
    x = x_ref[...]
    zg0 = zg_ref[:, :DIM]
    zg1 = zg_ref[:, DIM:]
    mi = mi_ref[0]

    def body(it, u):
        s0 = _dot8(zg0 * u, w)
        s1 = _dot8(zg1 * u, w)
        e0 = jnp.exp(s0)
        e1 = jnp.exp(s1)
        p0 = e0 / jnp.sum(e0, axis=1, keepdims=True)
        p1 = e1 / jnp.sum(e1, axis=1, keepdims=True)
        u2 = zg0 * _dot8(p0, wt) + zg1 * _dot8(p1, wt) + x
        un = _normseg(u2, w, wt)
        return jnp.where(it < mi - 1, un, u2)

    u1 = (zg0 + zg1) * 0.125 + x
    u1 = jnp.where(mi > 1, _normseg(u1, w, wt), u1)
    u1 = jnp.where(mi >= 1, u1, jnp.zeros((bn, DIM), jnp.float32))
    o_ref[...] = lax.fori_loop(1, mi, body, u1)


def _routing(z, zg2, mi_arr):
    n = z.shape[0]
    bn = 2000
    return pl.pallas_call(
        functools.partial(_route_body, bn=bn),
        grid=(n // bn,),
        in_specs=[
            pl.BlockSpec(memory_space=pltpu.SMEM),
            pl.BlockSpec((bn, DIM), lambda i: (i, 0)),
            pl.BlockSpec((bn, 2 * DIM), lambda i: (i, 0)),
        ],
        out_specs=pl.BlockSpec((bn, DIM), lambda i: (i, 0)),
        out_shape=jax.ShapeDtypeStruct((n, DIM), jnp.float32),
    )(mi_arr, z, zg2)


def kernel(input_, neighbors, max_iter):
    a, b, d = input_.shape
    n = a * b
    x = input_.reshape(n, d)
    z = _normalize(x)
    e_total = neighbors.shape[0]
    idx = neighbors.astype(jnp.int32) + (
        jnp.arange(e_total, dtype=jnp.int32) // 20) * 5
    zg = _gather(z, idx)
    zg2 = zg.reshape(n, 2 * d)
    mi_arr = jnp.asarray(max_iter, dtype=jnp.int32).reshape(1)
    u = _routing(z, zg2, mi_arr)
    return u.reshape(a, b, d)


# speedup vs baseline: 9.6121x; 1.0680x over previous
import functools

import jax
import jax.numpy as jnp
from jax import lax
from jax.experimental import pallas as pl
from jax.experimental.pallas import tpu as pltpu
from jax.experimental.pallas import tpu_sc as plsc

DIM = 256
K = 8
DD = DIM // K

NC = 2
NS = 16
NW = NC * NS
CH = 80


def _seg_matrix():
    r = lax.broadcasted_iota(jnp.int32, (DIM, K), 0) // DD
    c = lax.broadcasted_iota(jnp.int32, (DIM, K), 1)
    return (r == c).astype(jnp.float32)


def _seg_matrix_t():
    r = lax.broadcasted_iota(jnp.int32, (K, DIM), 0)
    c = lax.broadcasted_iota(jnp.int32, (K, DIM), 1) // DD
    return (r == c).astype(jnp.float32)


def _dot8(t, w):
    return lax.dot_general(t, w, (((1,), (0,)), ((), ())),
                           preferred_element_type=jnp.float32)


def _normseg(t, w, wt):
    n2 = _dot8(t * t, w)
    inv = 1.0 / jnp.maximum(jnp.sqrt(n2), 1e-12)
    return t * _dot8(inv, wt)


def _norm_body(x_ref, o_ref):
    o_ref[...] = _normseg(x_ref[...], _seg_matrix(), _seg_matrix_t())


def _normalize(x):
    n = x.shape[0]
    bn = 4000
    return pl.pallas_call(
        _norm_body,
        grid=(n // bn,),
        in_specs=[pl.BlockSpec((bn, DIM), lambda i: (i, 0))],
        out_specs=pl.BlockSpec((bn, DIM), lambda i: (i, 0)),
        out_shape=jax.ShapeDtypeStruct((n, DIM), jnp.float32),
    )(x)


def _gather(z, idx):
    e_total = idx.shape[0]
    n_chunks = e_total // CH
    t_max = (n_chunks + NW - 1) // NW
    mesh = plsc.VectorSubcoreMesh(
        core_axis_name="c", subcore_axis_name="s",
        num_cores=NC, num_subcores=NS)

    @functools.partial(
        pl.kernel,
        mesh=mesh,
        out_type=jax.ShapeDtypeStruct((e_total, DIM), jnp.float32),
        scratch_types=[
            pltpu.VMEM((CH,), jnp.int32),
            pltpu.VMEM((CH, DIM), jnp.float32),
            pltpu.SemaphoreType.DMA,
        ],
    )
    def k(z_hbm, idx_hbm, out_hbm, idx_v, rows_v, sem):
        wid = lax.axis_index("s") * NC + lax.axis_index("c")

        def body(t, carry):
            cid = t * NW + wid

            @pl.when(cid < n_chunks)
            def _():
                base = cid * CH
                pltpu.sync_copy(idx_hbm.at[pl.ds(base, CH)], idx_v)
                pltpu.async_copy(z_hbm.at[idx_v], rows_v, sem).wait()
                pltpu.sync_copy(rows_v, out_hbm.at[pl.ds(base, CH)])

            return carry

        lax.fori_loop(0, t_max, body, 0)

    return k(z, idx)


def _route_body(mi_ref, x_ref, zg_ref, o_ref, *, bn):
    w = _seg_matrix()
    wt = _seg_matrix_t()
    x = x_ref[...]
    zg0 = zg_ref[:, :DIM]
    zg1 = zg_ref[:, DIM:]
    mi = mi_ref[0]

    def body(it, u):
        s0 = _dot8(zg0 * u, w)
        s1 = _dot8(zg1 * u, w)
        e0 = jnp.exp(s0)
        e1 = jnp.exp(s1)
        p0 = e0 / jnp.sum(e0, axis=1, keepdims=True)
        p1 = e1 / jnp.sum(e1, axis=1, keepdims=True)
        u2 = zg0 * _dot8(p0, wt) + zg1 * _dot8(p1, wt) + x
        un = _normseg(u2, w, wt)
        return jnp.where(it < mi - 1, un, u2)

    u1 = (zg0 + zg1) * 0.125 + x
    u1 = jnp.where(mi > 1, _normseg(u1, w, wt), u1)
    u1 = jnp.where(mi >= 1, u1, jnp.zeros((bn, DIM), jnp.float32))
    o_ref[...] = lax.fori_loop(1, mi, body, u1)


def _route_piece_body(mi_ref, x_ref, zg_ref, ubuf_ref, o_ref, *, bn):
    del ubuf_ref
    _route_body(mi_ref, x_ref, zg_ref, o_ref, bn=bn)


def _routing_piece(z, zg2, ubuf, mi_arr, off_blocks):
    n = z.shape[0]
    bn = 2000
    npiece = zg2.shape[0]
    return pl.pallas_call(
        functools.partial(_route_piece_body, bn=bn),
        grid=(npiece // bn,),
        in_specs=[
            pl.BlockSpec(memory_space=pltpu.SMEM),
            pl.BlockSpec((bn, DIM), lambda i: (i + off_blocks, 0)),
            pl.BlockSpec((bn, 2 * DIM), lambda i: (i, 0)),
            pl.BlockSpec(memory_space=pl.ANY),
        ],
        out_specs=pl.BlockSpec((bn, DIM), lambda i: (i + off_blocks, 0)),
        out_shape=jax.ShapeDtypeStruct((n, DIM), jnp.float32),
        input_output_aliases={3: 0},
    )(mi_arr, z, zg2, ubuf)


def kernel(input_, neighbors, max_iter):
    a, b, d = input_.shape
    n = a * b
    x = input_.reshape(n, d)
    z = _normalize(x)
    e_total = neighbors.shape[0]
    idx = neighbors.astype(jnp.int32) + (
        jnp.arange(e_total, dtype=jnp.int32) // 20) * 5
    nsplit = 2
    e_piece = e_total // nsplit
    n_piece = n // nsplit
    zgs = [_gather(z, lax.slice(idx, (h * e_piece,), ((h + 1) * e_piece,)))
           for h in range(nsplit)]
    mi_arr = jnp.asarray(max_iter, dtype=jnp.int32).reshape(1)
    u = jnp.zeros((n, d), jnp.float32)
    for h in range(nsplit):
        u = _routing_piece(z, zgs[h].reshape(n_piece, 2 * d), u, mi_arr,
                           off_blocks=h * (n_piece // 2000))
    return u.reshape(a, b, d)


# speedup vs baseline: 11.7812x; 1.2257x over previous
import functools

import jax
import jax.numpy as jnp
from jax import lax
from jax.experimental import pallas as pl
from jax.experimental.pallas import tpu as pltpu
from jax.experimental.pallas import tpu_sc as plsc

DIM = 256
K = 8
DD = DIM // K

NC = 2
NS = 16
NW = NC * NS
CH = 80


def _seg_matrix():
    r = lax.broadcasted_iota(jnp.int32, (DIM, K), 0) // DD
    c = lax.broadcasted_iota(jnp.int32, (DIM, K), 1)
    return (r == c).astype(jnp.float32)


def _seg_matrix_t():
    r = lax.broadcasted_iota(jnp.int32, (K, DIM), 0)
    c = lax.broadcasted_iota(jnp.int32, (K, DIM), 1) // DD
    return (r == c).astype(jnp.float32)


def _dot8(t, w):
    return lax.dot_general(t, w, (((1,), (0,)), ((), ())),
                           preferred_element_type=jnp.float32)


def _pack_bf16(t):
    bits = lax.bitcast_convert_type(t, jnp.uint32)
    r = (bits + jnp.uint32(0x7FFF) + ((bits >> 16) & jnp.uint32(1))) >> 16
    half = t.shape[1] // 2
    word = r[:, :half] | (r[:, half:] << 16)
    return lax.bitcast_convert_type(word, jnp.int32)


def _unpack_bf16(wv):
    wu = lax.bitcast_convert_type(wv, jnp.uint32)
    lo = lax.bitcast_convert_type(wu << 16, jnp.float32)
    hi = lax.bitcast_convert_type(wu & jnp.uint32(0xFFFF0000), jnp.float32)
    return jnp.concatenate([lo, hi], axis=1)


def _normseg(t, w, wt):
    n2 = _dot8(t * t, w)
    inv = 1.0 / jnp.maximum(jnp.sqrt(n2), 1e-12)
    return t * _dot8(inv, wt)


def _norm_body(x_ref, o_ref):
    zn = _normseg(x_ref[...], _seg_matrix(), _seg_matrix_t())
    o_ref[...] = _pack_bf16(zn)


def _normalize(x):
    n = x.shape[0]
    bn = 4000
    return pl.pallas_call(
        _norm_body,
        grid=(n // bn,),
        in_specs=[pl.BlockSpec((bn, DIM), lambda i: (i, 0))],
        out_specs=pl.BlockSpec((bn, DIM // 2), lambda i: (i, 0)),
        out_shape=jax.ShapeDtypeStruct((n, DIM // 2), jnp.int32),
    )(x)


def _gather(z, idx):
    e_total = idx.shape[0]
    n_chunks = e_total // CH
    t_max = (n_chunks + NW - 1) // NW
    mesh = plsc.VectorSubcoreMesh(
        core_axis_name="c", subcore_axis_name="s",
        num_cores=NC, num_subcores=NS)

    @functools.partial(
        pl.kernel,
        mesh=mesh,
        out_type=jax.ShapeDtypeStruct((e_total, DIM // 2), jnp.int32),
        scratch_types=[
            pltpu.VMEM((CH,), jnp.int32),
            pltpu.VMEM((CH, DIM // 2), jnp.int32),
            pltpu.SemaphoreType.DMA,
        ],
    )
    def k(z_hbm, idx_hbm, out_hbm, idx_v, rows_v, sem):
        wid = lax.axis_index("s") * NC + lax.axis_index("c")

        def body(t, carry):
            cid = t * NW + wid

            @pl.when(cid < n_chunks)
            def _():
                base = cid * CH
                pltpu.sync_copy(idx_hbm.at[pl.ds(base, CH)], idx_v)
                pltpu.async_copy(z_hbm.at[idx_v], rows_v, sem).wait()
                pltpu.sync_copy(rows_v, out_hbm.at[pl.ds(base, CH)])

            return carry

        lax.fori_loop(0, t_max, body, 0)

    return k(z, idx)


def _route_body(mi_ref, x_ref, zg_ref, o_ref, *, bn):
    w = _seg_matrix()
    wt = _seg_matrix_t()
    x = _unpack_bf16(x_ref[...])
    zg0 = _unpack_bf16(zg_ref[:, :DIM // 2])
    zg1 = _unpack_bf16(zg_ref[:, DIM // 2:])
    mi = mi_ref[0]

    def body(it, u):
        s0 = _dot8(zg0 * u, w)
        s1 = _dot8(zg1 * u, w)
        e0 = jnp.exp(s0)
        e1 = jnp.exp(s1)
        p0 = e0 / jnp.sum(e0, axis=1, keepdims=True)
        p1 = e1 / jnp.sum(e1, axis=1, keepdims=True)
        u2 = zg0 * _dot8(p0, wt) + zg1 * _dot8(p1, wt) + x
        un = _normseg(u2, w, wt)
        return jnp.where(it < mi - 1, un, u2)

    u1 = (zg0 + zg1) * 0.125 + x
    u1 = jnp.where(mi > 1, _normseg(u1, w, wt), u1)
    u1 = jnp.where(mi >= 1, u1, jnp.zeros((bn, DIM), jnp.float32))
    o_ref[...] = lax.fori_loop(1, mi, body, u1)


def _route_piece_body(mi_ref, x_ref, zg_ref, ubuf_ref, o_ref, *, bn):
    del ubuf_ref
    _route_body(mi_ref, x_ref, zg_ref, o_ref, bn=bn)


def _routing_piece(z, zg2, ubuf, mi_arr, off_blocks):
    n = z.shape[0]
    bn = 2000
    npiece = zg2.shape[0]
    return pl.pallas_call(
        functools.partial(_route_piece_body, bn=bn),
        grid=(npiece // bn,),
        in_specs=[
            pl.BlockSpec(memory_space=pltpu.SMEM),
            pl.BlockSpec((bn, DIM // 2), lambda i: (i + off_blocks, 0)),
            pl.BlockSpec((bn, DIM), lambda i: (i, 0)),
            pl.BlockSpec(memory_space=pl.ANY),
        ],
        out_specs=pl.BlockSpec((bn, DIM), lambda i: (i + off_blocks, 0)),
        out_shape=jax.ShapeDtypeStruct((n, DIM), jnp.float32),
        input_output_aliases={3: 0},
    )(mi_arr, z, zg2, ubuf)


def kernel(input_, neighbors, max_iter):
    a, b, d = input_.shape
    n = a * b
    x = input_.reshape(n, d)
    z = _normalize(x)
    e_total = neighbors.shape[0]
    idx = neighbors.astype(jnp.int32) + (
        jnp.arange(e_total, dtype=jnp.int32) // 20) * 5
    nsplit = 2
    e_piece = e_total // nsplit
    n_piece = n // nsplit
    zgs = [_gather(z, lax.slice(idx, (h * e_piece,), ((h + 1) * e_piece,)))
           for h in range(nsplit)]
    mi_arr = jnp.asarray(max_iter, dtype=jnp.int32).reshape(1)
    u = jnp.zeros((n, d), jnp.float32)
    for h in range(nsplit):
        u = _routing_piece(z, zgs[h].reshape(n_piece, d), u, mi_arr,
                           off_blocks=h * (n_piece // 2000))
    return u.reshape(a, b, d)


# speedup vs baseline: 13.1041x; 1.1123x over previous
import functools

import jax
import jax.numpy as jnp
from jax import lax
from jax.experimental import pallas as pl
from jax.experimental.pallas import tpu as pltpu
from jax.experimental.pallas import tpu_sc as plsc

DIM = 256
K = 8
DD = DIM // K

NC = 2
NS = 16
NW = NC * NS
CH = 80


def _seg_matrix():
    r = lax.broadcasted_iota(jnp.int32, (DIM, K), 0) // DD
    c = lax.broadcasted_iota(jnp.int32, (DIM, K), 1)
    return (r == c).astype(jnp.float32)


def _seg_matrix_t():
    r = lax.broadcasted_iota(jnp.int32, (K, DIM), 0)
    c = lax.broadcasted_iota(jnp.int32, (K, DIM), 1) // DD
    return (r == c).astype(jnp.float32)


def _dot8(t, w):
    return lax.dot_general(t, w, (((1,), (0,)), ((), ())),
                           preferred_element_type=jnp.float32)


def _pack_bf16(t):
    bits = lax.bitcast_convert_type(t, jnp.uint32)
    r = (bits + jnp.uint32(0x7FFF) + ((bits >> 16) & jnp.uint32(1))) >> 16
    half = t.shape[1] // 2
    word = r[:, :half] | (r[:, half:] << 16)
    return lax.bitcast_convert_type(word, jnp.int32)


def _unpack_bf16(wv):
    wu = lax.bitcast_convert_type(wv, jnp.uint32)
    lo = lax.bitcast_convert_type(wu << 16, jnp.float32)
    hi = lax.bitcast_convert_type(wu & jnp.uint32(0xFFFF0000), jnp.float32)
    return jnp.concatenate([lo, hi], axis=1)


def _normseg(t, w, wt):
    n2 = _dot8(t * t, w)
    inv = 1.0 / jnp.maximum(jnp.sqrt(n2), 1e-12)
    return t * _dot8(inv, wt)


def _norm_body(x_ref, o_ref):
    zn = _normseg(x_ref[...], _seg_matrix(), _seg_matrix_t())
    o_ref[...] = _pack_bf16(zn)


def _normalize(x):
    n = x.shape[0]
    bn = 4000
    return pl.pallas_call(
        _norm_body,
        grid=(n // bn,),
        in_specs=[pl.BlockSpec((bn, DIM), lambda i: (i, 0))],
        out_specs=pl.BlockSpec((bn, DIM // 2), lambda i: (i, 0)),
        out_shape=jax.ShapeDtypeStruct((n, DIM // 2), jnp.int32),
    )(x)


def _gather(z, idx):
    e_total = idx.shape[0]
    n_chunks = e_total // CH
    t_max = (n_chunks + NW - 1) // NW
    mesh = plsc.VectorSubcoreMesh(
        core_axis_name="c", subcore_axis_name="s",
        num_cores=NC, num_subcores=NS)

    @functools.partial(
        pl.kernel,
        mesh=mesh,
        out_type=jax.ShapeDtypeStruct((e_total, DIM // 2), jnp.int32),
        scratch_types=[
            pltpu.VMEM((CH,), jnp.int32),
            pltpu.VMEM((CH, DIM // 2), jnp.int32),
            pltpu.SemaphoreType.DMA,
        ],
    )
    def k(z_hbm, idx_hbm, out_hbm, idx_v, rows_v, sem):
        wid = lax.axis_index("s") * NC + lax.axis_index("c")

        def body(t, carry):
            cid = t * NW + wid

            @pl.when(cid < n_chunks)
            def _():
                base = cid * CH
                pltpu.sync_copy(idx_hbm.at[pl.ds(base, CH)], idx_v)
                pltpu.async_copy(z_hbm.at[idx_v], rows_v, sem).wait()
                pltpu.sync_copy(rows_v, out_hbm.at[pl.ds(base, CH)])

            return carry

        lax.fori_loop(0, t_max, body, 0)

    return k(z, idx)


def _route_body(mi_ref, x_ref, zg_ref, o_ref, *, bn):
    w = _seg_matrix()
    wt = _seg_matrix_t()
    x = _unpack_bf16(x_ref[...])
    zg0 = _unpack_bf16(zg_ref[:, :DIM // 2])
    zg1 = _unpack_bf16(zg_ref[:, DIM // 2:])
    mi = mi_ref[0]

    def step(u):
        s0 = _dot8(zg0 * u, w)
        s1 = _dot8(zg1 * u, w)
        e0 = jnp.exp(s0)
        e1 = jnp.exp(s1)
        p0 = e0 / jnp.sum(e0, axis=1, keepdims=True)
        p1 = e1 / jnp.sum(e1, axis=1, keepdims=True)
        return zg0 * _dot8(p0, wt) + zg1 * _dot8(p1, wt) + x

    u1 = (zg0 + zg1) * 0.125 + x
    u1 = jnp.where(mi > 1, _normseg(u1, w, wt), u1)
    u1 = jnp.where(mi >= 1, u1, jnp.zeros((bn, DIM), jnp.float32))
    u = lax.fori_loop(1, mi - 1, lambda it, uu: _normseg(step(uu), w, wt), u1)
    u_fin = step(u)
    o_ref[...] = jnp.where(mi >= 2, u_fin, u)


def _route_piece_body(mi_ref, x_ref, zg_ref, ubuf_ref, o_ref, *, bn):
    del ubuf_ref
    _route_body(mi_ref, x_ref, zg_ref, o_ref, bn=bn)


BN_R = 1000


def _routing_piece(z, zg2, ubuf, mi_arr, off_blocks):
    n = z.shape[0]
    bn = BN_R
    npiece = zg2.shape[0]
    return pl.pallas_call(
        functools.partial(_route_piece_body, bn=bn),
        grid=(npiece // bn,),
        in_specs=[
            pl.BlockSpec(memory_space=pltpu.SMEM),
            pl.BlockSpec((bn, DIM // 2), lambda i: (i + off_blocks, 0)),
            pl.BlockSpec((bn, DIM), lambda i: (i, 0)),
            pl.BlockSpec(memory_space=pl.ANY),
        ],
        out_specs=pl.BlockSpec((bn, DIM), lambda i: (i + off_blocks, 0)),
        out_shape=jax.ShapeDtypeStruct((n, DIM), jnp.float32),
        input_output_aliases={3: 0},
    )(mi_arr, z, zg2, ubuf)


def kernel(input_, neighbors, max_iter):
    a, b, d = input_.shape
    n = a * b
    x = input_.reshape(n, d)
    z = _normalize(x)
    e_total = neighbors.shape[0]
    idx = neighbors.astype(jnp.int32) + (
        jnp.arange(e_total, dtype=jnp.int32) // 20) * 5
    nsplit = 4
    e_piece = e_total // nsplit
    n_piece = n // nsplit
    zgs = [_gather(z, lax.slice(idx, (h * e_piece,), ((h + 1) * e_piece,)))
           for h in range(nsplit)]
    mi_arr = jnp.asarray(max_iter, dtype=jnp.int32).reshape(1)
    u = jnp.zeros((n, d), jnp.float32)
    for h in range(nsplit):
        u = _routing_piece(z, zgs[h].reshape(n_piece, d), u, mi_arr,
                           off_blocks=h * (n_piece // BN_R))
    return u.reshape(a, b, d)


# speedup vs baseline: 13.5571x; 1.0346x over previous
import functools

import jax
import jax.numpy as jnp
from jax import lax
from jax.experimental import pallas as pl
from jax.experimental.pallas import tpu as pltpu
from jax.experimental.pallas import tpu_sc as plsc

DIM = 256
K = 8
DD = DIM // K

NC = 2
NS = 16
NW = NC * NS
CH = 80


def _seg_matrix():
    r = lax.broadcasted_iota(jnp.int32, (DIM, K), 0) // DD
    c = lax.broadcasted_iota(jnp.int32, (DIM, K), 1)
    return (r == c).astype(jnp.float32)


def _seg_matrix_t():
    r = lax.broadcasted_iota(jnp.int32, (K, DIM), 0)
    c = lax.broadcasted_iota(jnp.int32, (K, DIM), 1) // DD
    return (r == c).astype(jnp.float32)


def _dot8(t, w):
    return lax.dot_general(t, w, (((1,), (0,)), ((), ())),
                           preferred_element_type=jnp.float32)


def _pack_bf16(t):
    bits = lax.bitcast_convert_type(t, jnp.uint32)
    r = (bits + jnp.uint32(0x7FFF) + ((bits >> 16) & jnp.uint32(1))) >> 16
    half = t.shape[1] // 2
    word = r[:, :half] | (r[:, half:] << 16)
    return lax.bitcast_convert_type(word, jnp.int32)


def _unpack_bf16(wv):
    wu = lax.bitcast_convert_type(wv, jnp.uint32)
    lo = lax.bitcast_convert_type(wu << 16, jnp.float32)
    hi = lax.bitcast_convert_type(wu & jnp.uint32(0xFFFF0000), jnp.float32)
    return jnp.concatenate([lo, hi], axis=1)


def _normseg(t, w, wt):
    n2 = _dot8(t * t, w)
    inv = 1.0 / jnp.maximum(jnp.sqrt(n2), 1e-12)
    return t * _dot8(inv, wt)


def _norm_body(x_ref, o_ref):
    zn = _normseg(x_ref[...], _seg_matrix(), _seg_matrix_t())
    o_ref[...] = _pack_bf16(zn)


def _normalize(x):
    n = x.shape[0]
    bn = 4000
    return pl.pallas_call(
        _norm_body,
        grid=(n // bn,),
        in_specs=[pl.BlockSpec((bn, DIM), lambda i: (i, 0))],
        out_specs=pl.BlockSpec((bn, DIM // 2), lambda i: (i, 0)),
        out_shape=jax.ShapeDtypeStruct((n, DIM // 2), jnp.int32),
    )(x)


def _gather(z, idx):
    e_total = idx.shape[0]
    n_chunks = e_total // CH
    t_max = (n_chunks + NW - 1) // NW
    mesh = plsc.VectorSubcoreMesh(
        core_axis_name="c", subcore_axis_name="s",
        num_cores=NC, num_subcores=NS)

    @functools.partial(
        pl.kernel,
        mesh=mesh,
        out_type=jax.ShapeDtypeStruct((e_total, DIM // 2), jnp.int32),
        scratch_types=[
            pltpu.VMEM((CH,), jnp.int32),
            pltpu.VMEM((CH, DIM // 2), jnp.int32),
            pltpu.SemaphoreType.DMA,
        ],
    )
    def k(z_hbm, idx_hbm, out_hbm, idx_v, rows_v, sem):
        wid = lax.axis_index("s") * NC + lax.axis_index("c")

        def body(t, carry):
            cid = t * NW + wid

            @pl.when(cid < n_chunks)
            def _():
                base = cid * CH
                pltpu.sync_copy(idx_hbm.at[pl.ds(base, CH)], idx_v)
                pltpu.async_copy(z_hbm.at[idx_v], rows_v, sem).wait()
                pltpu.sync_copy(rows_v, out_hbm.at[pl.ds(base, CH)])

            return carry

        lax.fori_loop(0, t_max, body, 0)

    return k(z, idx)


def _route_body(mi_ref, x_ref, zg_ref, o_ref, *, bn):
    bf = jnp.bfloat16
    w = _seg_matrix().astype(bf)
    wt = _seg_matrix_t().astype(bf)
    x = _unpack_bf16(x_ref[...]).astype(bf)
    zg0 = _unpack_bf16(zg_ref[:, :DIM // 2]).astype(bf)
    zg1 = _unpack_bf16(zg_ref[:, DIM // 2:]).astype(bf)
    mi = mi_ref[0]

    def dotseg(t):
        return lax.dot_general(t, w, (((1,), (0,)), ((), ())),
                               preferred_element_type=jnp.float32)

    def bcast(s):
        r = lax.dot_general(s.astype(bf), wt, (((1,), (0,)), ((), ())),
                            preferred_element_type=jnp.float32)
        return r.astype(bf)

    def normseg16(t):
        n2 = dotseg(t * t)
        inv = 1.0 / jnp.maximum(jnp.sqrt(n2), 1e-12)
        return t * bcast(inv)

    def step(u):
        s0 = dotseg(zg0 * u)
        s1 = dotseg(zg1 * u)
        e0 = jnp.exp(s0)
        e1 = jnp.exp(s1)
        p0 = e0 / jnp.sum(e0, axis=1, keepdims=True)
        p1 = e1 / jnp.sum(e1, axis=1, keepdims=True)
        return zg0 * bcast(p0) + zg1 * bcast(p1) + x

    u1 = (zg0 + zg1) * bf(0.125) + x
    u1 = jnp.where(mi > 1, normseg16(u1), u1)
    u1 = jnp.where(mi >= 1, u1, jnp.zeros((bn, DIM), bf))
    u = lax.fori_loop(1, mi - 1, lambda it, uu: normseg16(step(uu)), u1)
    u_fin = step(u)
    o_ref[...] = jnp.where(mi >= 2, u_fin, u).astype(jnp.float32)


def _route_piece_body(mi_ref, x_ref, zg_ref, ubuf_ref, o_ref, *, bn):
    del ubuf_ref
    _route_body(mi_ref, x_ref, zg_ref, o_ref, bn=bn)


BN_R = 1000


def _routing_piece(z, zg2, ubuf, mi_arr, off_blocks):
    n = z.shape[0]
    bn = BN_R
    npiece = zg2.shape[0]
    return pl.pallas_call(
        functools.partial(_route_piece_body, bn=bn),
        grid=(npiece // bn,),
        in_specs=[
            pl.BlockSpec(memory_space=pltpu.SMEM),
            pl.BlockSpec((bn, DIM // 2), lambda i: (i + off_blocks, 0)),
            pl.BlockSpec((bn, DIM), lambda i: (i, 0)),
            pl.BlockSpec(memory_space=pl.ANY),
        ],
        out_specs=pl.BlockSpec((bn, DIM), lambda i: (i + off_blocks, 0)),
        out_shape=jax.ShapeDtypeStruct((n, DIM), jnp.float32),
        input_output_aliases={3: 0},
    )(mi_arr, z, zg2, ubuf)


def kernel(input_, neighbors, max_iter):
    a, b, d = input_.shape
    n = a * b
    x = input_.reshape(n, d)
    z = _normalize(x)
    e_total = neighbors.shape[0]
    idx = neighbors.astype(jnp.int32) + (
        jnp.arange(e_total, dtype=jnp.int32) // 20) * 5
    nsplit = 4
    e_piece = e_total // nsplit
    n_piece = n // nsplit
    zgs = [_gather(z, lax.slice(idx, (h * e_piece,), ((h + 1) * e_piece,)))
           for h in range(nsplit)]
    mi_arr = jnp.asarray(max_iter, dtype=jnp.int32).reshape(1)
    u = jnp.zeros((n, d), jnp.float32)
    for h in range(nsplit):
        u = _routing_piece(z, zgs[h].reshape(n_piece, d), u, mi_arr,
                           off_blocks=h * (n_piece // BN_R))
    return u.reshape(a, b, d)


# speedup vs baseline: 13.6138x; 1.0042x over previous
import functools

import jax
import jax.numpy as jnp
from jax import lax
from jax.experimental import pallas as pl
from jax.experimental.pallas import tpu as pltpu
from jax.experimental.pallas import tpu_sc as plsc

DIM = 256
K = 8
DD = DIM // K

NC = 2
NS = 16
NW = NC * NS
CH = 80


def _seg_matrix():
    r = lax.broadcasted_iota(jnp.int32, (DIM, K), 0) // DD
    c = lax.broadcasted_iota(jnp.int32, (DIM, K), 1)
    return (r == c).astype(jnp.float32)


def _seg_matrix_t():
    r = lax.broadcasted_iota(jnp.int32, (K, DIM), 0)
    c = lax.broadcasted_iota(jnp.int32, (K, DIM), 1) // DD
    return (r == c).astype(jnp.float32)


def _dot8(t, w):
    return lax.dot_general(t, w, (((1,), (0,)), ((), ())),
                           preferred_element_type=jnp.float32)


def _pack_bf16(t):
    bits = lax.bitcast_convert_type(t, jnp.uint32)
    r = (bits + jnp.uint32(0x7FFF) + ((bits >> 16) & jnp.uint32(1))) >> 16
    half = t.shape[1] // 2
    word = r[:, :half] | (r[:, half:] << 16)
    return lax.bitcast_convert_type(word, jnp.int32)


def _unpack_bf16(wv):
    wu = lax.bitcast_convert_type(wv, jnp.uint32)
    lo = lax.bitcast_convert_type(wu << 16, jnp.float32)
    hi = lax.bitcast_convert_type(wu & jnp.uint32(0xFFFF0000), jnp.float32)
    return jnp.concatenate([lo, hi], axis=1)


def _normseg(t, w, wt):
    n2 = _dot8(t * t, w)
    inv = 1.0 / jnp.maximum(jnp.sqrt(n2), 1e-12)
    return t * _dot8(inv, wt)


def _norm_body(x_ref, o_ref):
    zn = _normseg(x_ref[...], _seg_matrix(), _seg_matrix_t())
    o_ref[...] = _pack_bf16(zn)


def _normalize(x):
    n = x.shape[0]
    bn = 4000
    return pl.pallas_call(
        _norm_body,
        grid=(n // bn,),
        in_specs=[pl.BlockSpec((bn, DIM), lambda i: (i, 0))],
        out_specs=pl.BlockSpec((bn, DIM // 2), lambda i: (i, 0)),
        out_shape=jax.ShapeDtypeStruct((n, DIM // 2), jnp.int32),
    )(x)


def _gather(z, idx):
    e_total = idx.shape[0]
    n_chunks = e_total // CH
    t_max = (n_chunks + NW - 1) // NW
    mesh = plsc.VectorSubcoreMesh(
        core_axis_name="c", subcore_axis_name="s",
        num_cores=NC, num_subcores=NS)

    @functools.partial(
        pl.kernel,
        mesh=mesh,
        out_type=jax.ShapeDtypeStruct((e_total, DIM // 2), jnp.int32),
        scratch_types=[
            pltpu.VMEM((CH,), jnp.int32),
            pltpu.VMEM((CH, DIM // 2), jnp.int32),
            pltpu.SemaphoreType.DMA,
        ],
    )
    def k(z_hbm, idx_hbm, out_hbm, idx_v, rows_v, sem):
        wid = lax.axis_index("s") * NC + lax.axis_index("c")

        def body(t, carry):
            cid = t * NW + wid

            @pl.when(cid < n_chunks)
            def _():
                base = cid * CH
                pltpu.sync_copy(idx_hbm.at[pl.ds(base, CH)], idx_v)
                pltpu.async_copy(z_hbm.at[idx_v], rows_v, sem).wait()
                pltpu.sync_copy(rows_v, out_hbm.at[pl.ds(base, CH)])

            return carry

        lax.fori_loop(0, t_max, body, 0)

    return k(z, idx)


def _route_body(mi_ref, x_ref, zg_ref, o_ref, *, bn):
    bf = jnp.bfloat16
    w = _seg_matrix().astype(bf)
    wt = _seg_matrix_t().astype(bf)
    x = _unpack_bf16(x_ref[...]).astype(bf)
    zg0 = _unpack_bf16(zg_ref[:, :DIM // 2]).astype(bf)
    zg1 = _unpack_bf16(zg_ref[:, DIM // 2:]).astype(bf)
    mi = mi_ref[0]

    def dotseg(t):
        return lax.dot_general(t, w, (((1,), (0,)), ((), ())),
                               preferred_element_type=jnp.float32)

    def bcast(s):
        r = lax.dot_general(s.astype(bf), wt, (((1,), (0,)), ((), ())),
                            preferred_element_type=jnp.float32)
        return r.astype(bf)

    def normseg16(t):
        n2 = dotseg(t * t)
        inv = 1.0 / jnp.maximum(jnp.sqrt(n2), 1e-12)
        return t * bcast(inv)

    def step(u):
        s0 = dotseg(zg0 * u)
        s1 = dotseg(zg1 * u)
        e0 = jnp.exp(s0)
        e1 = jnp.exp(s1)
        p0 = e0 / jnp.sum(e0, axis=1, keepdims=True)
        p1 = e1 / jnp.sum(e1, axis=1, keepdims=True)
        return zg0 * bcast(p0) + zg1 * bcast(p1) + x

    u1 = (zg0 + zg1) * bf(0.125) + x
    u1 = jnp.where(mi > 1, normseg16(u1), u1)
    u1 = jnp.where(mi >= 1, u1, jnp.zeros((bn, DIM), bf))
    u = lax.fori_loop(1, mi - 1, lambda it, uu: normseg16(step(uu)), u1)
    u_fin = step(u)
    o_ref[...] = jnp.where(mi >= 2, u_fin, u).astype(jnp.float32)


def _route_piece_body(mi_ref, x_ref, zg_ref, ubuf_ref, o_ref, *, bn):
    del ubuf_ref
    _route_body(mi_ref, x_ref, zg_ref, o_ref, bn=bn)


BN_R = 1000


def _routing_piece(z, zg2, ubuf, mi_arr, off_blocks):
    n = z.shape[0]
    bn = BN_R
    npiece = zg2.shape[0]
    return pl.pallas_call(
        functools.partial(_route_piece_body, bn=bn),
        grid=(npiece // bn,),
        in_specs=[
            pl.BlockSpec(memory_space=pltpu.SMEM),
            pl.BlockSpec((bn, DIM // 2), lambda i: (i + off_blocks, 0)),
            pl.BlockSpec((bn, DIM), lambda i: (i, 0)),
            pl.BlockSpec(memory_space=pl.ANY),
        ],
        out_specs=pl.BlockSpec((bn, DIM), lambda i: (i + off_blocks, 0)),
        out_shape=jax.ShapeDtypeStruct((n, DIM), jnp.float32),
        input_output_aliases={3: 0},
    )(mi_arr, z, zg2, ubuf)


def kernel(input_, neighbors, max_iter):
    a, b, d = input_.shape
    n = a * b
    x = input_.reshape(n, d)
    z = _normalize(x)
    e_total = neighbors.shape[0]
    idx = neighbors.astype(jnp.int32) + (
        jnp.arange(e_total, dtype=jnp.int32) // 20) * 5
    nsplit = 5
    e_piece = e_total // nsplit
    n_piece = n // nsplit
    zgs = [_gather(z, lax.slice(idx, (h * e_piece,), ((h + 1) * e_piece,)))
           for h in range(nsplit)]
    mi_arr = jnp.asarray(max_iter, dtype=jnp.int32).reshape(1)
    u = jnp.zeros((n, d), jnp.float32)
    for h in range(nsplit):
        u = _routing_piece(z, zgs[h].reshape(n_piece, d), u, mi_arr,
                           off_blocks=h * (n_piece // BN_R))
    return u.reshape(a, b, d)
